# baseline (speedup 1.0000x reference)
import functools

import jax
import jax.numpy as jnp
from jax import lax
from jax.experimental import pallas as pl
from jax.experimental.pallas import tpu as pltpu
from jax.experimental.pallas import tpu_sc as plsc

N = 10000
E = 320000
D = 128
H = 64
NC = 2
NS = 16
NW = NC * NS
EPT = E // NW
B = 80
NB = EPT // B
ROW = 80
RPT = N // NS
UNROLL = 16
INV_SQRT_D = float(1.0 / (float(D) ** 0.5))
_UNPACK = functools.partial(plsc.unpack, format=plsc.PackFormat.INTERLEAVED)


def _qkv_body(x_ref, wq, bq, wk, bk, wv, bv, ws, bs,
              q_ref, k_ref, vlo_ref, vhi_ref, skip_ref):
    x = x_ref[...]
    q = jnp.dot(x, wq[...], preferred_element_type=jnp.float32) + bq[...]
    k = jnp.dot(x, wk[...], preferred_element_type=jnp.float32) + bk[...]
    v = jnp.dot(x, wv[...], preferred_element_type=jnp.float32) + bv[...]
    s = jnp.dot(x, ws[...], preferred_element_type=jnp.float32) + bs[...]
    q_ref[...] = q.astype(jnp.bfloat16)
    k_ref[...] = k.astype(jnp.bfloat16)
    vlo_ref[...] = v[:, :H].astype(jnp.bfloat16)
    vhi_ref[...] = v[:, H:].astype(jnp.bfloat16)
    skip_ref[...] = s


def _qkv(xx, wq, bq, wk, bk, wv, bv, ws, bs):
    bn = 1000
    wspec = pl.BlockSpec((D, D), lambda i: (0, 0))
    bspec = pl.BlockSpec((1, D), lambda i: (0, 0))
    return pl.pallas_call(
        _qkv_body,
        grid=(N // bn,),
        in_specs=[pl.BlockSpec((bn, D), lambda i: (i, 0)),
                  wspec, bspec, wspec, bspec, wspec, bspec, wspec, bspec],
        out_specs=[pl.BlockSpec((bn, D), lambda i: (i, 0)),
                   pl.BlockSpec((bn, D), lambda i: (i, 0)),
                   pl.BlockSpec((bn, H), lambda i: (i, 0)),
                   pl.BlockSpec((bn, H), lambda i: (i, 0)),
                   pl.BlockSpec((bn, D), lambda i: (i, 0))],
        out_shape=[jax.ShapeDtypeStruct((N, D), jnp.bfloat16),
                   jax.ShapeDtypeStruct((N, D), jnp.bfloat16),
                   jax.ShapeDtypeStruct((N, H), jnp.bfloat16),
                   jax.ShapeDtypeStruct((N, H), jnp.bfloat16),
                   jax.ShapeDtypeStruct((N, D), jnp.float32)],
    )(xx, wq, bq, wk, bk, wv, bv, ws, bs)


def _unperm(p):
    bn = p.shape[0]
    parts = []
    for c in range(H // 32):
        ev = p[:, 32 * c:32 * c + 16]
        od = p[:, 32 * c + 16:32 * c + 32]
        parts.append(jnp.stack([ev, od], axis=-1).reshape(bn, 32))
    return jnp.concatenate(parts, axis=1)


def _combine_body(lo0_ref, lo1_ref, hi0_ref, hi1_ref, skip_ref, o_ref, *, relu):
    plo = lo0_ref[...] + lo1_ref[...]
    phi = hi0_ref[...] + hi1_ref[...]
    den = jnp.maximum(plo[:, H:H + 1], 1e-16)
    agg = jnp.concatenate([_unperm(plo[:, :H]), _unperm(phi[:, :H])], axis=1)
    h = agg / den + skip_ref[...]
    if relu:
        h = jnp.maximum(h, 0.0)
    o_ref[...] = h


def _combine(lo, hi, skip, relu):
    bn = 1000
    pspec = pl.BlockSpec((bn, ROW), lambda i: (i, 0))
    return pl.pallas_call(
        functools.partial(_combine_body, relu=relu),
        grid=(N // bn,),
        in_specs=[pspec, pspec, pspec, pspec,
                  pl.BlockSpec((bn, D), lambda i: (i, 0))],
        out_specs=pl.BlockSpec((bn, D), lambda i: (i, 0)),
        out_shape=jax.ShapeDtypeStruct((N, D), jnp.float32),
    )(lo[0], lo[1], hi[0], hi[1], skip)


def _edge_body(q_hbm, k_hbm, vlo_hbm, vhi_hbm, src_hbm, dst_hbm, zrs_hbm,
               lo_hbm, hi_hbm,
               srcv, dstv, qrb, krb, vrb, wbuf, ob, tb,
               sem_q0, sem_q1, sem_k0, sem_k1, sem_v0, sem_v1,
               sem_s0, sem_s1, acc):
    cid = lax.axis_index("c")
    sid = lax.axis_index("s")
    wid = sid * NC + cid
    zero16 = jnp.zeros((16,), jnp.float32)
    lane = lax.broadcasted_iota(jnp.int32, (16,), 0)
    row0 = sid * RPT
    sem_q = (sem_q0, sem_q1)
    sem_k = (sem_k0, sem_k1)
    sem_v = (sem_v0, sem_v1)
    sem_s = (sem_s0, sem_s1)

    def zero_acc():
        pltpu.sync_copy(zrs_hbm, acc.at[pl.ds(row0, RPT)])

    def writeout(dst_ref):
        pltpu.sync_copy(acc.at[pl.ds(row0, RPT)],
                        dst_ref.at[cid, pl.ds(row0, RPT)])

    def g_issue_lo(b, p):
        pltpu.async_copy(q_hbm.at[dstv.at[b]], qrb.at[p], sem_q[p])
        pltpu.async_copy(k_hbm.at[srcv.at[b]], krb.at[p], sem_k[p])
        pltpu.async_copy(vlo_hbm.at[srcv.at[b]], vrb.at[p], sem_v[p])

    def g_wait_lo(b, p):
        pltpu.make_async_copy(q_hbm.at[dstv.at[b]], qrb.at[p], sem_q[p]).wait()
        pltpu.make_async_copy(k_hbm.at[srcv.at[b]], krb.at[p], sem_k[p]).wait()
        pltpu.make_async_copy(vlo_hbm.at[srcv.at[b]], vrb.at[p], sem_v[p]).wait()

    def g_issue_hi(b, p):
        pltpu.async_copy(vhi_hbm.at[srcv.at[b]], vrb.at[p], sem_v[p])

    def g_wait_hi(b, p):
        pltpu.make_async_copy(vhi_hbm.at[srcv.at[b]], vrb.at[p], sem_v[p]).wait()

    def s_issue(b, p):
        pltpu.async_copy(ob.at[p], acc.at[dstv.at[b]], sem_s[p], add=True)

    def s_wait(p):
        pltpu.make_async_copy(ob.at[p], acc.at[dstv.at[0]], sem_s[p]).wait()

    def compute_lo(b, p):
        @plsc.parallel_loop(0, B // UNROLL)
        def edges(i):
            par = i
            for l in range(UNROLL):
                e = i * UNROLL + l
                ts = []
                for j in range(D // 32):
                    qa, qx = _UNPACK(qrb[p, e, pl.ds(j * 32, 32)])
                    ka, kx = _UNPACK(krb[p, e, pl.ds(j * 32, 32)])
                    ts.append(qa * ka + qx * kx)
                tb[par, l, pl.ds(0, 16)] = (ts[0] + ts[1]) + (ts[2] + ts[3])
            cols = [plsc.load_gather(tb.at[par],
                                     [lane, jnp.full((16,), c, jnp.int32)])
                    for c in range(16)]
            while len(cols) > 1:
                cols = [cols[k] + cols[k + 1] for k in range(0, len(cols), 2)]
            w16 = jnp.exp(cols[0] * INV_SQRT_D)
            wbuf[b, pl.ds(i * UNROLL, 16)] = w16
            for l in range(UNROLL):
                e = i * UNROLL + l
                w = jnp.full((16,), w16[l], jnp.float32)
                ob[p, e, pl.ds(H, 16)] = jnp.where(lane == 0, w, zero16)
                for j in range(H // 32):
                    va, vx = _UNPACK(vrb[p, e, pl.ds(j * 32, 32)])
                    ob[p, e, pl.ds(j * 32, 16)] = w * va
                    ob[p, e, pl.ds(j * 32 + 16, 16)] = w * vx

    def compute_hi(b, p):
        @plsc.parallel_loop(0, B // UNROLL)
        def edges(i):
            wvec = wbuf[b, pl.ds(i * UNROLL, 16)]
            for l in range(UNROLL):
                e = i * UNROLL + l
                w = jnp.full((16,), wvec[l], jnp.float32)
                ob[p, e, pl.ds(H, 16)] = zero16
                for j in range(H // 32):
                    va, vx = _UNPACK(vrb[p, e, pl.ds(j * 32, 32)])
                    ob[p, e, pl.ds(j * 32, 16)] = w * va
                    ob[p, e, pl.ds(j * 32 + 16, 16)] = w * vx

    def run_phase(g_issue, g_wait, compute):
        g_issue(0, 0)
        g_issue(1, 1)

        def pair(i, c):
            b0 = 2 * i
            b1 = b0 + 1
            g_wait(b0, 0)

            @pl.when(i >= 1)
            def _():
                s_wait(0)
            compute(b0, 0)
            s_issue(b0, 0)
            g_issue(b0 + 2, 0)

            g_wait(b1, 1)

            @pl.when(i >= 1)
            def _():
                s_wait(1)
            compute(b1, 1)
            s_issue(b1, 1)

            @pl.when(i < (NB - 3) // 2)
            def _():
                g_issue(b1 + 2, 1)
            return c
        lax.fori_loop(0, (NB - 1) // 2, pair, 0)

        bl = NB - 1
        g_wait(bl, 0)
        s_wait(0)
        compute(bl, 0)
        s_issue(bl, 0)
        s_wait(0)
        s_wait(1)

    zero_acc()
    plsc.subcore_barrier()

    pltpu.sync_copy(src_hbm.at[wid], srcv)
    pltpu.sync_copy(dst_hbm.at[wid], dstv)

    run_phase(g_issue_lo, g_wait_lo, compute_lo)
    plsc.subcore_barrier()

    writeout(lo_hbm)
    zero_acc()
    plsc.subcore_barrier()

    run_phase(g_issue_hi, g_wait_hi, compute_hi)
    plsc.subcore_barrier()

    writeout(hi_hbm)


_edge = pl.kernel(
    _edge_body,
    out_type=(jax.ShapeDtypeStruct((NC, N, ROW), jnp.float32),
              jax.ShapeDtypeStruct((NC, N, ROW), jnp.float32)),
    mesh=plsc.VectorSubcoreMesh(core_axis_name="c", subcore_axis_name="s"),
    scratch_types=[
        pltpu.VMEM((NB, B), jnp.int32),
        pltpu.VMEM((NB, B), jnp.int32),
        pltpu.VMEM((2, B, D), jnp.bfloat16),
        pltpu.VMEM((2, B, D), jnp.bfloat16),
        pltpu.VMEM((2, B, H), jnp.bfloat16),
        pltpu.VMEM((NB, B), jnp.float32),
        pltpu.VMEM((2, B, ROW), jnp.float32),
        pltpu.VMEM((B // UNROLL, 16, 16), jnp.float32),
        pltpu.SemaphoreType.DMA,
        pltpu.SemaphoreType.DMA,
        pltpu.SemaphoreType.DMA,
        pltpu.SemaphoreType.DMA,
        pltpu.SemaphoreType.DMA,
        pltpu.SemaphoreType.DMA,
        pltpu.SemaphoreType.DMA,
        pltpu.SemaphoreType.DMA,
        pltpu.VMEM_SHARED((N, ROW), jnp.float32),
    ],
    compiler_params=pltpu.CompilerParams(use_tc_tiling_on_sc=False,
                                         needs_layout_passes=False),
)


def kernel(x, edge_index, Wq1, bq1, Wk1, bk1, Wv1, bv1, Ws1, bs1,
           Wq2, bq2, Wk2, bk2, Wv2, bv2, Ws2, bs2):
    src = edge_index[0].reshape(NW, NB, B)
    dst = edge_index[1].reshape(NW, NB, B)
    zrs = jnp.zeros((RPT, ROW), jnp.float32)
    b2 = lambda b: b.reshape(1, D)

    q1, k1, vlo1, vhi1, skip1 = _qkv(x, Wq1, b2(bq1), Wk1, b2(bk1),
                                     Wv1, b2(bv1), Ws1, b2(bs1))
    lo1, hi1 = _edge(q1, k1, vlo1, vhi1, src, dst, zrs)
    h = _combine(lo1, hi1, skip1, relu=True)

    q2, k2, vlo2, vhi2, skip2 = _qkv(h, Wq2, b2(bq2), Wk2, b2(bk2),
                                     Wv2, b2(bv2), Ws2, b2(bs2))
    lo2, hi2 = _edge(q2, k2, vlo2, vhi2, src, dst, zrs)
    return _combine(lo2, hi2, skip2, relu=False)


# speedup vs baseline: 1.2829x; 1.2829x over previous
import functools

import jax
import jax.numpy as jnp
from jax import lax
from jax.experimental import pallas as pl
from jax.experimental.pallas import tpu as pltpu
from jax.experimental.pallas import tpu_sc as plsc

N = 10000
E = 320000
D = 128
H = 64
NC = 2
NS = 16
NW = NC * NS
EPT = E // NW
B = 80
NB = EPT // B
ROW = 80
RPT = N // NS
UNROLL = 16
INV_SQRT_D = float(1.0 / (float(D) ** 0.5))
_UNPACK = functools.partial(plsc.unpack, format=plsc.PackFormat.INTERLEAVED)


def _qkv_body(x_ref, wq, bq, wk, bk, wv, bv, ws, bs,
              q_ref, k_ref, vlo_ref, vhi_ref, skip_ref):
    x = x_ref[...]
    q = jnp.dot(x, wq[...], preferred_element_type=jnp.float32) + bq[...]
    k = jnp.dot(x, wk[...], preferred_element_type=jnp.float32) + bk[...]
    v = jnp.dot(x, wv[...], preferred_element_type=jnp.float32) + bv[...]
    s = jnp.dot(x, ws[...], preferred_element_type=jnp.float32) + bs[...]
    q_ref[...] = q.astype(jnp.bfloat16)
    k_ref[...] = k.astype(jnp.bfloat16)
    vlo_ref[...] = v[:, :H]
    vhi_ref[...] = v[:, H:]
    skip_ref[...] = s


def _qkv(xx, wq, bq, wk, bk, wv, bv, ws, bs):
    bn = 1000
    wspec = pl.BlockSpec((D, D), lambda i: (0, 0))
    bspec = pl.BlockSpec((1, D), lambda i: (0, 0))
    return pl.pallas_call(
        _qkv_body,
        grid=(N // bn,),
        in_specs=[pl.BlockSpec((bn, D), lambda i: (i, 0)),
                  wspec, bspec, wspec, bspec, wspec, bspec, wspec, bspec],
        out_specs=[pl.BlockSpec((bn, D), lambda i: (i, 0)),
                   pl.BlockSpec((bn, D), lambda i: (i, 0)),
                   pl.BlockSpec((bn, H), lambda i: (i, 0)),
                   pl.BlockSpec((bn, H), lambda i: (i, 0)),
                   pl.BlockSpec((bn, D), lambda i: (i, 0))],
        out_shape=[jax.ShapeDtypeStruct((N, D), jnp.bfloat16),
                   jax.ShapeDtypeStruct((N, D), jnp.bfloat16),
                   jax.ShapeDtypeStruct((N, H), jnp.float32),
                   jax.ShapeDtypeStruct((N, H), jnp.float32),
                   jax.ShapeDtypeStruct((N, D), jnp.float32)],
    )(xx, wq, bq, wk, bk, wv, bv, ws, bs)


def _combine_body(lo0_ref, lo1_ref, hi0_ref, hi1_ref, skip_ref, o_ref, *, relu):
    plo = lo0_ref[...] + lo1_ref[...]
    phi = hi0_ref[...] + hi1_ref[...]
    den = jnp.maximum(plo[:, H:H + 1], 1e-16)
    agg = jnp.concatenate([plo[:, :H], phi[:, :H]], axis=1)
    h = agg / den + skip_ref[...]
    if relu:
        h = jnp.maximum(h, 0.0)
    o_ref[...] = h


def _combine(lo, hi, skip, relu):
    bn = 1000
    pspec = pl.BlockSpec((bn, ROW), lambda i: (i, 0))
    return pl.pallas_call(
        functools.partial(_combine_body, relu=relu),
        grid=(N // bn,),
        in_specs=[pspec, pspec, pspec, pspec,
                  pl.BlockSpec((bn, D), lambda i: (i, 0))],
        out_specs=pl.BlockSpec((bn, D), lambda i: (i, 0)),
        out_shape=jax.ShapeDtypeStruct((N, D), jnp.float32),
    )(lo[0], lo[1], hi[0], hi[1], skip)


def _mid_body(lo0_ref, lo1_ref, hi0_ref, hi1_ref, skip_ref,
              wq, bq, wk, bk, wv, bv, ws, bs,
              q_ref, k_ref, vlo_ref, vhi_ref, skip2_ref):
    plo = lo0_ref[...] + lo1_ref[...]
    phi = hi0_ref[...] + hi1_ref[...]
    den = jnp.maximum(plo[:, H:H + 1], 1e-16)
    agg = jnp.concatenate([plo[:, :H], phi[:, :H]], axis=1)
    h = jnp.maximum(agg / den + skip_ref[...], 0.0)
    q = jnp.dot(h, wq[...], preferred_element_type=jnp.float32) + bq[...]
    k = jnp.dot(h, wk[...], preferred_element_type=jnp.float32) + bk[...]
    v = jnp.dot(h, wv[...], preferred_element_type=jnp.float32) + bv[...]
    s2 = jnp.dot(h, ws[...], preferred_element_type=jnp.float32) + bs[...]
    q_ref[...] = q.astype(jnp.bfloat16)
    k_ref[...] = k.astype(jnp.bfloat16)
    vlo_ref[...] = v[:, :H]
    vhi_ref[...] = v[:, H:]
    skip2_ref[...] = s2


def _mid(lo, hi, skip, wq, bq, wk, bk, wv, bv, ws, bs):
    bn = 1000
    pspec = pl.BlockSpec((bn, ROW), lambda i: (i, 0))
    wspec = pl.BlockSpec((D, D), lambda i: (0, 0))
    bspec = pl.BlockSpec((1, D), lambda i: (0, 0))
    return pl.pallas_call(
        _mid_body,
        grid=(N // bn,),
        in_specs=[pspec, pspec, pspec, pspec,
                  pl.BlockSpec((bn, D), lambda i: (i, 0)),
                  wspec, bspec, wspec, bspec, wspec, bspec, wspec, bspec],
        out_specs=[pl.BlockSpec((bn, D), lambda i: (i, 0)),
                   pl.BlockSpec((bn, D), lambda i: (i, 0)),
                   pl.BlockSpec((bn, H), lambda i: (i, 0)),
                   pl.BlockSpec((bn, H), lambda i: (i, 0)),
                   pl.BlockSpec((bn, D), lambda i: (i, 0))],
        out_shape=[jax.ShapeDtypeStruct((N, D), jnp.bfloat16),
                   jax.ShapeDtypeStruct((N, D), jnp.bfloat16),
                   jax.ShapeDtypeStruct((N, H), jnp.float32),
                   jax.ShapeDtypeStruct((N, H), jnp.float32),
                   jax.ShapeDtypeStruct((N, D), jnp.float32)],
    )(lo[0], lo[1], hi[0], hi[1], skip, wq, bq, wk, bk, wv, bv, ws, bs)


def _edge_body(q_hbm, k_hbm, vlo_hbm, vhi_hbm, src_hbm, dst_hbm, zrs_hbm,
               lo_hbm, hi_hbm,
               srcv, dstv, qrb, krb, vrb, wbuf, ob, tb,
               sem_q0, sem_q1, sem_k0, sem_k1, sem_v0, sem_v1,
               sem_s0, sem_s1, acc):
    cid = lax.axis_index("c")
    sid = lax.axis_index("s")
    wid = sid * NC + cid
    zero16 = jnp.zeros((16,), jnp.float32)
    lane = lax.broadcasted_iota(jnp.int32, (16,), 0)
    row0 = sid * RPT
    sem_q = (sem_q0, sem_q1)
    sem_k = (sem_k0, sem_k1)
    sem_v = (sem_v0, sem_v1)
    sem_s = (sem_s0, sem_s1)

    def zero_acc():
        pltpu.sync_copy(zrs_hbm, acc.at[pl.ds(row0, RPT)])

    def writeout(dst_ref):
        pltpu.sync_copy(acc.at[pl.ds(row0, RPT)],
                        dst_ref.at[cid, pl.ds(row0, RPT)])

    def g_issue_lo(b, p):
        pltpu.async_copy(q_hbm.at[dstv.at[b]], qrb.at[p], sem_q[p])
        pltpu.async_copy(k_hbm.at[srcv.at[b]], krb.at[p], sem_k[p])
        pltpu.async_copy(vlo_hbm.at[srcv.at[b]], vrb.at[p], sem_v[p])

    def g_wait_lo(b, p):
        pltpu.make_async_copy(q_hbm.at[dstv.at[b]], qrb.at[p], sem_q[p]).wait()
        pltpu.make_async_copy(k_hbm.at[srcv.at[b]], krb.at[p], sem_k[p]).wait()
        pltpu.make_async_copy(vlo_hbm.at[srcv.at[b]], vrb.at[p], sem_v[p]).wait()

    def g_issue_hi(b, p):
        pltpu.async_copy(vhi_hbm.at[srcv.at[b]], vrb.at[p], sem_v[p])

    def g_wait_hi(b, p):
        pltpu.make_async_copy(vhi_hbm.at[srcv.at[b]], vrb.at[p], sem_v[p]).wait()

    def s_issue(b, p):
        pltpu.async_copy(ob.at[p], acc.at[dstv.at[b]], sem_s[p], add=True)

    def s_wait(p):
        pltpu.make_async_copy(ob.at[p], acc.at[dstv.at[0]], sem_s[p]).wait()

    def compute_lo(b, p):
        @plsc.parallel_loop(0, B // UNROLL)
        def edges(i):
            par = i
            for l in range(UNROLL):
                e = i * UNROLL + l
                ts = []
                for j in range(D // 32):
                    qa, qx = _UNPACK(qrb[p, e, pl.ds(j * 32, 32)])
                    ka, kx = _UNPACK(krb[p, e, pl.ds(j * 32, 32)])
                    ts.append(qa * ka + qx * kx)
                tb[par, l, pl.ds(0, 16)] = (ts[0] + ts[1]) + (ts[2] + ts[3])
            cols = [plsc.load_gather(tb.at[par],
                                     [lane, jnp.full((16,), c, jnp.int32)])
                    for c in range(16)]
            while len(cols) > 1:
                cols = [cols[k] + cols[k + 1] for k in range(0, len(cols), 2)]
            w16 = jnp.exp(cols[0] * INV_SQRT_D)
            wbuf[b, pl.ds(i * UNROLL, 16)] = w16
            for l in range(UNROLL):
                e = i * UNROLL + l
                w = jnp.full((16,), w16[l], jnp.float32)
                ob[p, e, pl.ds(H, 16)] = jnp.where(lane == 0, w, zero16)
                for j in range(H // 16):
                    ob[p, e, pl.ds(j * 16, 16)] = w * vrb[p, e, pl.ds(j * 16, 16)]

    def compute_hi(b, p):
        @plsc.parallel_loop(0, B // UNROLL)
        def edges(i):
            wvec = wbuf[b, pl.ds(i * UNROLL, 16)]
            for l in range(UNROLL):
                e = i * UNROLL + l
                w = jnp.full((16,), wvec[l], jnp.float32)
                ob[p, e, pl.ds(H, 16)] = zero16
                for j in range(H // 16):
                    ob[p, e, pl.ds(j * 16, 16)] = w * vrb[p, e, pl.ds(j * 16, 16)]

    def run_phase(g_issue, g_wait, compute):
        g_issue(0, 0)
        g_issue(1, 1)

        def pair(i, c):
            b0 = 2 * i
            b1 = b0 + 1
            g_wait(b0, 0)

            @pl.when(i >= 1)
            def _():
                s_wait(0)
            compute(b0, 0)
            s_issue(b0, 0)
            g_issue(b0 + 2, 0)

            g_wait(b1, 1)

            @pl.when(i >= 1)
            def _():
                s_wait(1)
            compute(b1, 1)
            s_issue(b1, 1)

            @pl.when(i < (NB - 3) // 2)
            def _():
                g_issue(b1 + 2, 1)
            return c
        lax.fori_loop(0, (NB - 1) // 2, pair, 0)

        bl = NB - 1
        g_wait(bl, 0)
        s_wait(0)
        compute(bl, 0)
        s_issue(bl, 0)
        s_wait(0)
        s_wait(1)

    zero_acc()
    plsc.subcore_barrier()

    pltpu.sync_copy(src_hbm.at[wid], srcv)
    pltpu.sync_copy(dst_hbm.at[wid], dstv)

    run_phase(g_issue_lo, g_wait_lo, compute_lo)
    plsc.subcore_barrier()

    writeout(lo_hbm)
    zero_acc()
    plsc.subcore_barrier()

    run_phase(g_issue_hi, g_wait_hi, compute_hi)
    plsc.subcore_barrier()

    writeout(hi_hbm)


_edge = pl.kernel(
    _edge_body,
    out_type=(jax.ShapeDtypeStruct((NC, N, ROW), jnp.float32),
              jax.ShapeDtypeStruct((NC, N, ROW), jnp.float32)),
    mesh=plsc.VectorSubcoreMesh(core_axis_name="c", subcore_axis_name="s"),
    scratch_types=[
        pltpu.VMEM((NB, B), jnp.int32),
        pltpu.VMEM((NB, B), jnp.int32),
        pltpu.VMEM((2, B, D), jnp.bfloat16),
        pltpu.VMEM((2, B, D), jnp.bfloat16),
        pltpu.VMEM((2, B, H), jnp.float32),
        pltpu.VMEM((NB, B), jnp.float32),
        pltpu.VMEM((2, B, ROW), jnp.float32),
        pltpu.VMEM((B // UNROLL, 16, 16), jnp.float32),
        pltpu.SemaphoreType.DMA,
        pltpu.SemaphoreType.DMA,
        pltpu.SemaphoreType.DMA,
        pltpu.SemaphoreType.DMA,
        pltpu.SemaphoreType.DMA,
        pltpu.SemaphoreType.DMA,
        pltpu.SemaphoreType.DMA,
        pltpu.SemaphoreType.DMA,
        pltpu.VMEM_SHARED((N, ROW), jnp.float32),
    ],
    compiler_params=pltpu.CompilerParams(use_tc_tiling_on_sc=False,
                                         needs_layout_passes=False),
)


def kernel(x, edge_index, Wq1, bq1, Wk1, bk1, Wv1, bv1, Ws1, bs1,
           Wq2, bq2, Wk2, bk2, Wv2, bv2, Ws2, bs2):
    src = edge_index[0].reshape(NW, NB, B)
    dst = edge_index[1].reshape(NW, NB, B)
    zrs = jnp.zeros((RPT, ROW), jnp.float32)
    b2 = lambda b: b.reshape(1, D)

    q1, k1, vlo1, vhi1, skip1 = _qkv(x, Wq1, b2(bq1), Wk1, b2(bk1),
                                     Wv1, b2(bv1), Ws1, b2(bs1))
    lo1, hi1 = _edge(q1, k1, vlo1, vhi1, src, dst, zrs)
    q2, k2, vlo2, vhi2, skip2 = _mid(lo1, hi1, skip1, Wq2, b2(bq2),
                                     Wk2, b2(bk2), Wv2, b2(bv2),
                                     Ws2, b2(bs2))
    lo2, hi2 = _edge(q2, k2, vlo2, vhi2, src, dst, zrs)
    return _combine(lo2, hi2, skip2, relu=False)


# speedup vs baseline: 1.2853x; 1.0018x over previous
import functools

import jax
import jax.numpy as jnp
from jax import lax
from jax.experimental import pallas as pl
from jax.experimental.pallas import tpu as pltpu
from jax.experimental.pallas import tpu_sc as plsc

N = 10000
E = 320000
D = 128
H = 64
NC = 2
NS = 16
NW = NC * NS
EPT = E // NW
B = 80
NB = EPT // B
ROW = 80
RPT = N // NS
UNROLL = 16
INV_SQRT_D = float(1.0 / (float(D) ** 0.5))
_UNPACK = functools.partial(plsc.unpack, format=plsc.PackFormat.INTERLEAVED)


def _qkv_body(x_ref, wq, bq, wk, bk, wv, bv, ws, bs,
              q_ref, k_ref, vlo_ref, vhi_ref, skip_ref):
    x = x_ref[...]
    q = jnp.dot(x, wq[...], preferred_element_type=jnp.float32) + bq[...]
    k = jnp.dot(x, wk[...], preferred_element_type=jnp.float32) + bk[...]
    v = jnp.dot(x, wv[...], preferred_element_type=jnp.float32) + bv[...]
    s = jnp.dot(x, ws[...], preferred_element_type=jnp.float32) + bs[...]
    q_ref[...] = q.astype(jnp.bfloat16)
    k_ref[...] = k.astype(jnp.bfloat16)
    vlo_ref[...] = v[:, :H]
    vhi_ref[...] = v[:, H:]
    skip_ref[...] = s


def _qkv(xx, wq, bq, wk, bk, wv, bv, ws, bs):
    bn = 1000
    wspec = pl.BlockSpec((D, D), lambda i: (0, 0))
    bspec = pl.BlockSpec((1, D), lambda i: (0, 0))
    return pl.pallas_call(
        _qkv_body,
        grid=(N // bn,),
        in_specs=[pl.BlockSpec((bn, D), lambda i: (i, 0)),
                  wspec, bspec, wspec, bspec, wspec, bspec, wspec, bspec],
        out_specs=[pl.BlockSpec((bn, D), lambda i: (i, 0)),
                   pl.BlockSpec((bn, D), lambda i: (i, 0)),
                   pl.BlockSpec((bn, H), lambda i: (i, 0)),
                   pl.BlockSpec((bn, H), lambda i: (i, 0)),
                   pl.BlockSpec((bn, D), lambda i: (i, 0))],
        out_shape=[jax.ShapeDtypeStruct((N, D), jnp.bfloat16),
                   jax.ShapeDtypeStruct((N, D), jnp.bfloat16),
                   jax.ShapeDtypeStruct((N, H), jnp.float32),
                   jax.ShapeDtypeStruct((N, H), jnp.float32),
                   jax.ShapeDtypeStruct((N, D), jnp.float32)],
    )(xx, wq, bq, wk, bk, wv, bv, ws, bs)


def _combine_body(lo0_ref, lo1_ref, hi0_ref, hi1_ref, skip_ref, o_ref, *, relu):
    plo = lo0_ref[...] + lo1_ref[...]
    phi = hi0_ref[...] + hi1_ref[...]
    den = jnp.maximum(plo[:, H:H + 1], 1e-16)
    agg = jnp.concatenate([plo[:, :H], phi[:, :H]], axis=1)
    h = agg / den + skip_ref[...]
    if relu:
        h = jnp.maximum(h, 0.0)
    o_ref[...] = h


def _combine(lo, hi, skip, relu):
    bn = 1000
    pspec = pl.BlockSpec((bn, ROW), lambda i: (i, 0))
    return pl.pallas_call(
        functools.partial(_combine_body, relu=relu),
        grid=(N // bn,),
        in_specs=[pspec, pspec, pspec, pspec,
                  pl.BlockSpec((bn, D), lambda i: (i, 0))],
        out_specs=pl.BlockSpec((bn, D), lambda i: (i, 0)),
        out_shape=jax.ShapeDtypeStruct((N, D), jnp.float32),
    )(lo[0], lo[1], hi[0], hi[1], skip)


def _mid_body(lo0_ref, lo1_ref, hi0_ref, hi1_ref, skip_ref,
              wq, bq, wk, bk, wv, bv, ws, bs,
              q_ref, k_ref, vlo_ref, vhi_ref, skip2_ref):
    plo = lo0_ref[...] + lo1_ref[...]
    phi = hi0_ref[...] + hi1_ref[...]
    den = jnp.maximum(plo[:, H:H + 1], 1e-16)
    agg = jnp.concatenate([plo[:, :H], phi[:, :H]], axis=1)
    h = jnp.maximum(agg / den + skip_ref[...], 0.0)
    q = jnp.dot(h, wq[...], preferred_element_type=jnp.float32) + bq[...]
    k = jnp.dot(h, wk[...], preferred_element_type=jnp.float32) + bk[...]
    v = jnp.dot(h, wv[...], preferred_element_type=jnp.float32) + bv[...]
    s2 = jnp.dot(h, ws[...], preferred_element_type=jnp.float32) + bs[...]
    q_ref[...] = q.astype(jnp.bfloat16)
    k_ref[...] = k.astype(jnp.bfloat16)
    vlo_ref[...] = v[:, :H]
    vhi_ref[...] = v[:, H:]
    skip2_ref[...] = s2


def _mid(lo, hi, skip, wq, bq, wk, bk, wv, bv, ws, bs):
    bn = 1000
    pspec = pl.BlockSpec((bn, ROW), lambda i: (i, 0))
    wspec = pl.BlockSpec((D, D), lambda i: (0, 0))
    bspec = pl.BlockSpec((1, D), lambda i: (0, 0))
    return pl.pallas_call(
        _mid_body,
        grid=(N // bn,),
        in_specs=[pspec, pspec, pspec, pspec,
                  pl.BlockSpec((bn, D), lambda i: (i, 0)),
                  wspec, bspec, wspec, bspec, wspec, bspec, wspec, bspec],
        out_specs=[pl.BlockSpec((bn, D), lambda i: (i, 0)),
                   pl.BlockSpec((bn, D), lambda i: (i, 0)),
                   pl.BlockSpec((bn, H), lambda i: (i, 0)),
                   pl.BlockSpec((bn, H), lambda i: (i, 0)),
                   pl.BlockSpec((bn, D), lambda i: (i, 0))],
        out_shape=[jax.ShapeDtypeStruct((N, D), jnp.bfloat16),
                   jax.ShapeDtypeStruct((N, D), jnp.bfloat16),
                   jax.ShapeDtypeStruct((N, H), jnp.float32),
                   jax.ShapeDtypeStruct((N, H), jnp.float32),
                   jax.ShapeDtypeStruct((N, D), jnp.float32)],
    )(lo[0], lo[1], hi[0], hi[1], skip, wq, bq, wk, bk, wv, bv, ws, bs)


def _edge_body(q_hbm, k_hbm, vlo_hbm, vhi_hbm, src_hbm, dst_hbm, zrs_hbm,
               lo_hbm, hi_hbm,
               srcv, dstv, qrb, krb, vrb, wbuf, ob, tb,
               sem_q0, sem_q1, sem_k0, sem_k1, sem_v0, sem_v1,
               sem_s0, sem_s1, acc):
    cid = lax.axis_index("c")
    sid = lax.axis_index("s")
    wid = sid * NC + cid
    zero16 = jnp.zeros((16,), jnp.float32)
    lane = lax.broadcasted_iota(jnp.int32, (16,), 0)
    row0 = sid * RPT
    sem_q = (sem_q0, sem_q1)
    sem_k = (sem_k0, sem_k1)
    sem_v = (sem_v0, sem_v1)
    sem_s = (sem_s0, sem_s1)

    def zero_acc():
        pltpu.sync_copy(zrs_hbm, acc.at[pl.ds(row0, RPT)])

    def writeout(dst_ref):
        pltpu.sync_copy(acc.at[pl.ds(row0, RPT)],
                        dst_ref.at[cid, pl.ds(row0, RPT)])

    def g_issue_lo(b, p):
        pltpu.async_copy(q_hbm.at[dstv.at[b]], qrb.at[p], sem_q[p])
        pltpu.async_copy(k_hbm.at[srcv.at[b]], krb.at[p], sem_k[p])
        pltpu.async_copy(vlo_hbm.at[srcv.at[b]], vrb.at[p], sem_v[p])

    def g_wait_lo(b, p):
        pltpu.make_async_copy(q_hbm.at[dstv.at[b]], qrb.at[p], sem_q[p]).wait()
        pltpu.make_async_copy(k_hbm.at[srcv.at[b]], krb.at[p], sem_k[p]).wait()
        pltpu.make_async_copy(vlo_hbm.at[srcv.at[b]], vrb.at[p], sem_v[p]).wait()

    def g_issue_hi(b, p):
        pltpu.async_copy(vhi_hbm.at[srcv.at[b]], vrb.at[p], sem_v[p])

    def g_wait_hi(b, p):
        pltpu.make_async_copy(vhi_hbm.at[srcv.at[b]], vrb.at[p], sem_v[p]).wait()

    def s_issue(b, p):
        pltpu.async_copy(ob.at[p], acc.at[dstv.at[b]], sem_s[p], add=True)

    def s_wait(p):
        pltpu.make_async_copy(ob.at[p], acc.at[dstv.at[0]], sem_s[p]).wait()

    def compute_lo(b, p):
        @plsc.parallel_loop(0, B // UNROLL)
        def edges(i):
            par = i
            for l in range(UNROLL):
                e = i * UNROLL + l
                ts = []
                for j in range(D // 32):
                    qa, qx = _UNPACK(qrb[p, e, pl.ds(j * 32, 32)])
                    ka, kx = _UNPACK(krb[p, e, pl.ds(j * 32, 32)])
                    ts.append(qa * ka + qx * kx)
                tb[par, l, pl.ds(0, 16)] = (ts[0] + ts[1]) + (ts[2] + ts[3])
            cols = [plsc.load_gather(tb.at[par],
                                     [lane, jnp.full((16,), c, jnp.int32)])
                    for c in range(16)]
            while len(cols) > 1:
                cols = [cols[k] + cols[k + 1] for k in range(0, len(cols), 2)]
            w16 = jnp.exp(cols[0] * INV_SQRT_D)
            wbuf[b, pl.ds(i * UNROLL, 16)] = w16
            for l in range(UNROLL):
                e = i * UNROLL + l
                w = jnp.full((16,), w16[l], jnp.float32)
                ob[p, e, pl.ds(H, 16)] = jnp.where(lane == 0, w, zero16)
                for j in range(H // 16):
                    ob[p, e, pl.ds(j * 16, 16)] = w * vrb[p, e, pl.ds(j * 16, 16)]

    def compute_hi(b, p):
        @plsc.parallel_loop(0, B // UNROLL)
        def edges(i):
            wvec = wbuf[b, pl.ds(i * UNROLL, 16)]
            for l in range(UNROLL):
                e = i * UNROLL + l
                w = jnp.full((16,), wvec[l], jnp.float32)
                ob[p, e, pl.ds(H, 16)] = zero16
                for j in range(H // 16):
                    ob[p, e, pl.ds(j * 16, 16)] = w * vrb[p, e, pl.ds(j * 16, 16)]

    def run_phase(g_issue, g_wait, compute, pre=None, prologue=True):
        if prologue:
            g_issue(0, 0)
            g_issue(1, 1)

        def pair(i, c):
            b0 = 2 * i
            b1 = b0 + 1
            g_wait(b0, 0)

            @pl.when(i >= 1)
            def _():
                s_wait(0)
            compute(b0, 0)
            s_issue(b0, 0)
            g_issue(b0 + 2, 0)

            g_wait(b1, 1)

            @pl.when(i >= 1)
            def _():
                s_wait(1)
            compute(b1, 1)
            s_issue(b1, 1)

            @pl.when(i < (NB - 3) // 2)
            def _():
                g_issue(b1 + 2, 1)
            return c
        lax.fori_loop(0, (NB - 1) // 2, pair, 0)

        bl = NB - 1
        g_wait(bl, 0)
        s_wait(0)
        compute(bl, 0)
        s_issue(bl, 0)
        s_wait(0)
        s_wait(1)
        if pre is not None:
            pre()

    zero_acc()
    plsc.subcore_barrier()

    pltpu.sync_copy(src_hbm.at[wid], srcv)
    pltpu.sync_copy(dst_hbm.at[wid], dstv)

    def pre_hi():
        g_issue_hi(0, 0)
        g_issue_hi(1, 1)
    run_phase(g_issue_lo, g_wait_lo, compute_lo, pre=pre_hi)
    plsc.subcore_barrier()

    writeout(lo_hbm)
    zero_acc()
    plsc.subcore_barrier()

    run_phase(g_issue_hi, g_wait_hi, compute_hi, prologue=False)
    plsc.subcore_barrier()

    writeout(hi_hbm)


_edge = pl.kernel(
    _edge_body,
    out_type=(jax.ShapeDtypeStruct((NC, N, ROW), jnp.float32),
              jax.ShapeDtypeStruct((NC, N, ROW), jnp.float32)),
    mesh=plsc.VectorSubcoreMesh(core_axis_name="c", subcore_axis_name="s"),
    scratch_types=[
        pltpu.VMEM((NB, B), jnp.int32),
        pltpu.VMEM((NB, B), jnp.int32),
        pltpu.VMEM((2, B, D), jnp.bfloat16),
        pltpu.VMEM((2, B, D), jnp.bfloat16),
        pltpu.VMEM((2, B, H), jnp.float32),
        pltpu.VMEM((NB, B), jnp.float32),
        pltpu.VMEM((2, B, ROW), jnp.float32),
        pltpu.VMEM((B // UNROLL, 16, 16), jnp.float32),
        pltpu.SemaphoreType.DMA,
        pltpu.SemaphoreType.DMA,
        pltpu.SemaphoreType.DMA,
        pltpu.SemaphoreType.DMA,
        pltpu.SemaphoreType.DMA,
        pltpu.SemaphoreType.DMA,
        pltpu.SemaphoreType.DMA,
        pltpu.SemaphoreType.DMA,
        pltpu.VMEM_SHARED((N, ROW), jnp.float32),
    ],
    compiler_params=pltpu.CompilerParams(use_tc_tiling_on_sc=False,
                                         needs_layout_passes=False),
)


def kernel(x, edge_index, Wq1, bq1, Wk1, bk1, Wv1, bv1, Ws1, bs1,
           Wq2, bq2, Wk2, bk2, Wv2, bv2, Ws2, bs2):
    src = edge_index[0].reshape(NW, NB, B)
    dst = edge_index[1].reshape(NW, NB, B)
    zrs = jnp.zeros((RPT, ROW), jnp.float32)
    b2 = lambda b: b.reshape(1, D)

    q1, k1, vlo1, vhi1, skip1 = _qkv(x, Wq1, b2(bq1), Wk1, b2(bk1),
                                     Wv1, b2(bv1), Ws1, b2(bs1))
    lo1, hi1 = _edge(q1, k1, vlo1, vhi1, src, dst, zrs)
    q2, k2, vlo2, vhi2, skip2 = _mid(lo1, hi1, skip1, Wq2, b2(bq2),
                                     Wk2, b2(bk2), Wv2, b2(bv2),
                                     Ws2, b2(bs2))
    lo2, hi2 = _edge(q2, k2, vlo2, vhi2, src, dst, zrs)
    return _combine(lo2, hi2, skip2, relu=False)
